# Initial kernel scaffold; baseline (speedup 1.0000x reference)
import jax
import jax.numpy as jnp
from jax.experimental import pallas as pl


def kernel(h, edge_index, edge_weight, W0, b0, W1, b1, W2, b2):
    raise NotImplementedError("write your pallas kernel here")


# speedup vs baseline: 3.7725x; 3.7725x over previous
import functools

import jax
import jax.numpy as jnp
from jax import lax
from jax.experimental import pallas as pl
from jax.experimental.pallas import tpu as pltpu
from jax.experimental.pallas import tpu_sc as plsc

N = 10000
NP = 10240
E = 320000
D = 128
NC = 2
NS = 16
L = 16
NW = NC * NS
CH = 128
NCHUNK = E // CH
CPW = (NCHUNK + NW - 1) // NW
ROWS_PT = NP // NS

def _worker_id():
    return lax.axis_index("s") * NC + lax.axis_index("c")


def _mesh():
    return plsc.VectorSubcoreMesh(
        core_axis_name="c", subcore_axis_name="s", num_cores=NC, num_subcores=NS
    )


def _degree_body(src_hbm, dst_hbm, dego_hbm, degi_hbm,
                 dego_loc, degi_loc, sv, dv):
    wid = _worker_id()

    def z(i, _):
        dego_loc[pl.ds(i * L, L)] = jnp.zeros((L,), jnp.float32)
        degi_loc[pl.ds(i * L, L)] = jnp.zeros((L,), jnp.float32)
        return 0

    lax.fori_loop(0, N // L, z, 0)

    ones = jnp.ones((L,), jnp.float32)

    def chunk_body(i, _):
        c = i * NW + wid

        @pl.when(c < NCHUNK)
        def _():
            base = c * CH
            pltpu.sync_copy(src_hbm.at[pl.ds(base, CH)], sv)
            pltpu.sync_copy(dst_hbm.at[pl.ds(base, CH)], dv)
            for g in range(CH // L):
                plsc.addupdate_scatter(dego_loc, [sv[pl.ds(g * L, L)]], ones)
                plsc.addupdate_scatter(degi_loc, [dv[pl.ds(g * L, L)]], ones)

        return 0

    lax.fori_loop(0, CPW, chunk_body, 0)

    pltpu.sync_copy(dego_loc, dego_hbm.at[pl.ds(wid * N, N)])
    pltpu.sync_copy(degi_loc, degi_hbm.at[pl.ds(wid * N, N)])


def _edge_weight_body(src_hbm, dst_hbm, ew_hbm, no_hbm, ni_hbm, we_hbm,
                      no_v, ni_v, sv, dv, ewv, ov):
    wid = _worker_id()
    pltpu.sync_copy(no_hbm, no_v)
    pltpu.sync_copy(ni_hbm, ni_v)

    def chunk_body(i, _):
        c = i * NW + wid

        @pl.when(c < NCHUNK)
        def _():
            base = c * CH
            pltpu.sync_copy(src_hbm.at[pl.ds(base, CH)], sv)
            pltpu.sync_copy(dst_hbm.at[pl.ds(base, CH)], dv)
            pltpu.sync_copy(ew_hbm.at[pl.ds(base, CH)], ewv)
            for g in range(CH // L):
                s = sv[pl.ds(g * L, L)]
                d = dv[pl.ds(g * L, L)]
                w = ewv[pl.ds(g * L, L)]
                ov[pl.ds(g * L, L)] = (
                    w * plsc.load_gather(no_v, [s]) * plsc.load_gather(ni_v, [d])
                )
            pltpu.sync_copy(ov, we_hbm.at[pl.ds(base, CH)])

        return 0

    lax.fori_loop(0, CPW, chunk_body, 0)


def _spmm_body(x_hbm, src_hbm, dst_hbm, we_hbm, zeros_hbm, acc_hbm,
               sv, dv2, wev, rows, acc_sh, sem):
    cid = lax.axis_index("c")
    sid = lax.axis_index("s")
    wid = _worker_id()

    base_r = sid * ROWS_PT
    pltpu.sync_copy(zeros_hbm.at[pl.ds(base_r, ROWS_PT)],
                    acc_sh.at[pl.ds(base_r, ROWS_PT)])
    plsc.subcore_barrier()

    def chunk_body(i, _):
        c = i * NW + wid

        @pl.when(c < NCHUNK)
        def _():
            base = c * CH
            pltpu.sync_copy(src_hbm.at[pl.ds(base, CH)], sv)
            pltpu.sync_copy(we_hbm.at[pl.ds(base, CH)], wev)
            pltpu.sync_copy(dst_hbm.at[pl.ds(base, CH)], dv2.at[0])
            pltpu.async_copy(x_hbm.at[sv], rows, sem).wait()

            def scale(j, _):
                wb = plsc.load_gather(wev, [jnp.zeros((L,), jnp.int32) + j])
                for k in range(D // L):
                    rows[j, pl.ds(k * L, L)] = rows[j, pl.ds(k * L, L)] * wb
                return 0

            lax.fori_loop(0, CH, scale, 0)
            pltpu.sync_copy(rows, acc_sh.at[dv2.at[0]], add=True)

        return 0

    lax.fori_loop(0, CPW, chunk_body, 0)
    plsc.subcore_barrier()

    pltpu.sync_copy(acc_sh.at[pl.ds(base_r, ROWS_PT)],
                    acc_hbm.at[cid, pl.ds(base_r, ROWS_PT)])


@functools.lru_cache(maxsize=None)
def _sc_kernels():
    mesh = _mesh()
    degree = pl.kernel(
        _degree_body,
        out_type=(
            jax.ShapeDtypeStruct((NW * N,), jnp.float32),
            jax.ShapeDtypeStruct((NW * N,), jnp.float32),
        ),
        mesh=mesh,
        compiler_params=pltpu.CompilerParams(needs_layout_passes=False),
        scratch_types=(
            pltpu.VMEM((N,), jnp.float32),
            pltpu.VMEM((N,), jnp.float32),
            pltpu.VMEM((CH,), jnp.int32),
            pltpu.VMEM((CH,), jnp.int32),
        ),
    )
    edge_weight = pl.kernel(
        _edge_weight_body,
        out_type=jax.ShapeDtypeStruct((E,), jnp.float32),
        mesh=mesh,
        compiler_params=pltpu.CompilerParams(needs_layout_passes=False),
        scratch_types=(
            pltpu.VMEM((N,), jnp.float32),
            pltpu.VMEM((N,), jnp.float32),
            pltpu.VMEM((CH,), jnp.int32),
            pltpu.VMEM((CH,), jnp.int32),
            pltpu.VMEM((CH,), jnp.float32),
            pltpu.VMEM((CH,), jnp.float32),
        ),
    )
    spmm = pl.kernel(
        _spmm_body,
        out_type=jax.ShapeDtypeStruct((NC, NP, D), jnp.float32),
        mesh=mesh,
        compiler_params=pltpu.CompilerParams(needs_layout_passes=False),
        scratch_types=(
            pltpu.VMEM((CH,), jnp.int32),
            pltpu.VMEM((1, CH), jnp.int32),
            pltpu.VMEM((CH,), jnp.float32),
            pltpu.VMEM((CH, D), jnp.float32),
            pltpu.VMEM_SHARED((NP, D), jnp.float32),
            pltpu.SemaphoreType.DMA,
        ),
    )
    return degree, edge_weight, spmm


def _matmul(accp, Wm, bv, act):
    RB = 1000

    def body(a_ref, w_ref, b_ref, o_ref):
        acc = a_ref[0] + a_ref[1]
        y = jnp.dot(acc, w_ref[...], preferred_element_type=jnp.float32)
        y = y + b_ref[...]
        if act:
            y = jnp.where(y > 0, y, jnp.exp(y) - 1.0)
        o_ref[...] = y

    return pl.pallas_call(
        body,
        grid=(N // RB,),
        in_specs=[
            pl.BlockSpec((NC, RB, D), lambda i: (0, i, 0)),
            pl.BlockSpec((D, D), lambda i: (0, 0)),
            pl.BlockSpec((1, D), lambda i: (0, 0)),
        ],
        out_specs=pl.BlockSpec((RB, D), lambda i: (i, 0)),
        out_shape=jax.ShapeDtypeStruct((N, D), jnp.float32),
    )(accp, Wm, bv)


def kernel(h, edge_index, edge_weight, W0, b0, W1, b1, W2, b2):
    src = edge_index[0]
    dst = edge_index[1]
    _degree_kernel, _edge_weight_kernel, _spmm_kernel = _sc_kernels()

    dego_p, degi_p = _degree_kernel(src, dst)
    dego = dego_p.reshape(NW, N).sum(axis=0)
    degi = degi_p.reshape(NW, N).sum(axis=0)
    no = jnp.where(dego > 0, dego, 1.0) ** -0.5
    ni = jnp.where(degi > 0, degi, 1.0) ** -0.5

    we = _edge_weight_kernel(src, dst, edge_weight, no, ni)

    zeros = jnp.zeros((NP, D), jnp.float32)
    x = h
    for Wm, bv, act in ((W0, b0, True), (W1, b1, True), (W2, b2, False)):
        accp = _spmm_kernel(x, src, dst, we, zeros)
        x = _matmul(accp, Wm, bv.reshape(1, D), act)
    return x
